# baseline (device time: 25230 ns/iter reference)
import jax
import jax.numpy as jnp
from jax import lax
from jax.experimental import pallas as pl
from jax.experimental.pallas import tpu as pltpu

N_DEV = 4
TRAILER = 32


def kernel(x, w_mat):
    m_per, _ = x.shape
    _, n = w_mat.shape
    n_per = n // N_DEV

    def body(x_ref, w_ref, out_ref, send_buf, recv_buf, stage,
             send_sems, recv_sems, out_sems):
        my = lax.axis_index("i")

        barrier = pltpu.get_barrier_semaphore()
        for off in range(1, N_DEV):
            pl.semaphore_signal(
                barrier, inc=1,
                device_id=(lax.rem(my + off, N_DEV),),
                device_id_type=pl.DeviceIdType.MESH,
            )
        pl.semaphore_wait(barrier, N_DEV - 1)

        x_val = x_ref[...]
        out_copies = []

        for d in range(N_DEV):
            blk = jnp.dot(
                x_val, w_ref[:, d * n_per:(d + 1) * n_per],
                preferred_element_type=jnp.float32,
            )
            blk = jnp.maximum(blk, 0.0)

            @pl.when(my == d)
            def _():
                stage[d] = blk
                cp = pltpu.make_async_copy(
                    stage.at[d],
                    out_ref.at[pl.ds(d * m_per, m_per), :],
                    out_sems.at[d],
                )
                cp.start()

            @pl.when(my != d)
            def _():
                smax = jnp.maximum(jnp.max(blk), 1e-6)
                q = (blk * (127.0 / smax) + 0.5).astype(jnp.int8)
                send_buf[d, 0:m_per, :] = q
                scale_i8 = pltpu.bitcast(
                    jnp.full((8, n_per), smax, jnp.float32), jnp.int8
                )
                send_buf[d, m_per:m_per + TRAILER, :] = scale_i8
                send = pltpu.make_async_remote_copy(
                    src_ref=send_buf.at[d],
                    dst_ref=recv_buf.at[my],
                    send_sem=send_sems.at[d],
                    recv_sem=recv_sems.at[my],
                    device_id=(d,),
                    device_id_type=pl.DeviceIdType.MESH,
                )
                send.start()

        for s in range(N_DEV):
            @pl.when(my != s)
            def _():
                recv = pltpu.make_async_remote_copy(
                    src_ref=send_buf.at[s],
                    dst_ref=recv_buf.at[s],
                    send_sem=send_sems.at[s],
                    recv_sem=recv_sems.at[s],
                    device_id=(s,),
                    device_id_type=pl.DeviceIdType.MESH,
                )
                recv.wait_recv()
                scale = pltpu.bitcast(
                    recv_buf[s, m_per:m_per + TRAILER, :], jnp.float32
                )[0, 0]
                stage[s] = (
                    recv_buf[s, 0:m_per, :].astype(jnp.float32)
                    * (scale * (1.0 / 127.0))
                )
                cp = pltpu.make_async_copy(
                    stage.at[s],
                    out_ref.at[pl.ds(s * m_per, m_per), :],
                    out_sems.at[s],
                )
                cp.start()

        for d in range(N_DEV):
            cp = pltpu.make_async_copy(
                stage.at[d],
                out_ref.at[pl.ds(d * m_per, m_per), :],
                out_sems.at[d],
            )
            cp.wait()

            @pl.when(my != d)
            def _():
                send = pltpu.make_async_remote_copy(
                    src_ref=send_buf.at[d],
                    dst_ref=recv_buf.at[my],
                    send_sem=send_sems.at[d],
                    recv_sem=recv_sems.at[my],
                    device_id=(d,),
                    device_id_type=pl.DeviceIdType.MESH,
                )
                send.wait_send()

    return pl.pallas_call(
        body,
        out_shape=jax.ShapeDtypeStruct((N_DEV * m_per, n_per), jnp.float32),
        in_specs=[
            pl.BlockSpec(memory_space=pltpu.VMEM),
            pl.BlockSpec(memory_space=pltpu.VMEM),
        ],
        out_specs=pl.BlockSpec(memory_space=pl.ANY),
        scratch_shapes=[
            pltpu.VMEM((N_DEV, m_per + TRAILER, n_per), jnp.int8),
            pltpu.VMEM((N_DEV, m_per + TRAILER, n_per), jnp.int8),
            pltpu.VMEM((N_DEV, m_per, n_per), jnp.float32),
            pltpu.SemaphoreType.DMA((N_DEV,)),
            pltpu.SemaphoreType.DMA((N_DEV,)),
            pltpu.SemaphoreType.DMA((N_DEV,)),
        ],
        compiler_params=pltpu.CompilerParams(collective_id=0),
    )(x, w_mat)


# device time: 24394 ns/iter; 1.0343x vs baseline; 1.0343x over previous
import jax
import jax.numpy as jnp
from jax import lax
from jax.experimental import pallas as pl
from jax.experimental.pallas import tpu as pltpu

N_DEV = 4
TRAILER = 32


def kernel(x, w_mat):
    m_per, _ = x.shape
    _, n = w_mat.shape
    n_per = n // N_DEV

    def body(x_ref, w_ref, out_ref, send_buf, recv_buf, send_sems, recv_sems):
        my = lax.axis_index("i")

        barrier = pltpu.get_barrier_semaphore()
        for off in range(1, N_DEV):
            pl.semaphore_signal(
                barrier, inc=1,
                device_id=(lax.rem(my + off, N_DEV),),
                device_id_type=pl.DeviceIdType.MESH,
            )

        x_val = x_ref[...]

        for d in range(N_DEV):
            blk = jnp.dot(
                x_val, w_ref[:, d * n_per:(d + 1) * n_per],
                preferred_element_type=jnp.float32,
            )
            blk = jnp.maximum(blk, 0.0)
            if d == 0:
                pl.semaphore_wait(barrier, N_DEV - 1)

            @pl.when(my == d)
            def _():
                out_ref[pl.ds(my * m_per, m_per), :] = blk

            @pl.when(my != d)
            def _():
                col_max = jnp.maximum(
                    jnp.max(blk, axis=0, keepdims=True), 1e-6
                )
                q = (blk * (127.0 / col_max) + 0.5).astype(jnp.int8)
                send_buf[d, 0:m_per, :] = q
                scale_i8 = pltpu.bitcast(
                    jnp.broadcast_to(col_max, (8, n_per)), jnp.int8
                )
                send_buf[d, m_per:m_per + TRAILER, :] = scale_i8
                send = pltpu.make_async_remote_copy(
                    src_ref=send_buf.at[d],
                    dst_ref=recv_buf.at[my],
                    send_sem=send_sems.at[d],
                    recv_sem=recv_sems.at[my],
                    device_id=(d,),
                    device_id_type=pl.DeviceIdType.MESH,
                )
                send.start()

        for s in range(N_DEV):
            @pl.when(my != s)
            def _():
                recv = pltpu.make_async_remote_copy(
                    src_ref=send_buf.at[s],
                    dst_ref=recv_buf.at[s],
                    send_sem=send_sems.at[s],
                    recv_sem=recv_sems.at[s],
                    device_id=(s,),
                    device_id_type=pl.DeviceIdType.MESH,
                )
                recv.wait_recv()
                col_scale = pltpu.bitcast(
                    recv_buf[s, m_per:m_per + TRAILER, :], jnp.float32
                )[0:1, :] * (1.0 / 127.0)
                out_ref[pl.ds(s * m_per, m_per), :] = (
                    recv_buf[s, 0:m_per, :].astype(jnp.float32) * col_scale
                )

        for d in range(N_DEV):
            @pl.when(my != d)
            def _():
                send = pltpu.make_async_remote_copy(
                    src_ref=send_buf.at[d],
                    dst_ref=recv_buf.at[my],
                    send_sem=send_sems.at[d],
                    recv_sem=recv_sems.at[my],
                    device_id=(d,),
                    device_id_type=pl.DeviceIdType.MESH,
                )
                send.wait_send()

    return pl.pallas_call(
        body,
        out_shape=jax.ShapeDtypeStruct((N_DEV * m_per, n_per), jnp.float32),
        in_specs=[
            pl.BlockSpec(memory_space=pltpu.VMEM),
            pl.BlockSpec(memory_space=pltpu.VMEM),
        ],
        out_specs=pl.BlockSpec(memory_space=pltpu.VMEM),
        scratch_shapes=[
            pltpu.VMEM((N_DEV, m_per + TRAILER, n_per), jnp.int8),
            pltpu.VMEM((N_DEV, m_per + TRAILER, n_per), jnp.int8),
            pltpu.SemaphoreType.DMA((N_DEV,)),
            pltpu.SemaphoreType.DMA((N_DEV,)),
        ],
        compiler_params=pltpu.CompilerParams(collective_id=0),
    )(x, w_mat)
